# Initial kernel scaffold; baseline (speedup 1.0000x reference)
import jax
import jax.numpy as jnp
from jax.experimental import pallas as pl


def kernel(x, adj_mat, U0, b0, U, Q, P):
    raise NotImplementedError("write your pallas kernel here")


# speedup vs baseline: 1.2963x; 1.2963x over previous
import jax
import jax.numpy as jnp
from jax.experimental import pallas as pl
from jax.experimental.pallas import tpu as pltpu

_N = 10000
_D = 128
_IB = 2048
_KB = 1000
_NI = -(-_N // _IB)


def _proj_kernel(x_ref, U0_ref, b0_ref, h_ref):
    h_ref[...] = (
        jax.lax.dot_general(
            x_ref[...], U0_ref[...],
            dimension_numbers=(((1,), (0,)), ((), ())),
            preferred_element_type=jnp.float32,
            precision=jax.lax.Precision.HIGHEST,
        )
        + b0_ref[...]
    )


def _main_kernel(h_ref, adj_ref, U_ref, Q_ref, P_ref, out_ref,
                 S_acc, deg_acc, g_acc):
    i = pl.program_id(0)
    k = pl.program_id(1)
    ni = pl.num_programs(0)
    nk = pl.num_programs(1)

    @pl.when(k == 0)
    def _init():
        S_acc[...] = jnp.zeros_like(S_acc)
        deg_acc[...] = jnp.zeros_like(deg_acc)

    a = adj_ref[...].astype(jnp.bfloat16)
    hk = h_ref[pl.ds(k * _KB, _KB), :]
    hk_hi = hk.astype(jnp.bfloat16)
    hk_lo = (hk - hk_hi.astype(jnp.float32)).astype(jnp.bfloat16)
    S_acc[...] += jax.lax.dot_general(
        a, hk_hi, dimension_numbers=(((0,), (0,)), ((), ())),
        preferred_element_type=jnp.float32,
    ) + jax.lax.dot_general(
        a, hk_lo, dimension_numbers=(((0,), (0,)), ((), ())),
        preferred_element_type=jnp.float32,
    )
    ones = jnp.ones((_KB, 1), jnp.bfloat16)
    deg_acc[...] += jax.lax.dot_general(
        a, ones, dimension_numbers=(((0,), (0,)), ((), ())),
        preferred_element_type=jnp.float32,
    )

    @pl.when(k == nk - 1)
    def _epilogue():
        S = S_acc[...] / deg_acc[...]
        h2 = jnp.maximum(
            jax.lax.dot_general(
                S, U_ref[...], dimension_numbers=(((1,), (1,)), ((), ())),
                preferred_element_type=jnp.float32,
                precision=jax.lax.Precision.HIGHEST,
            ),
            0.0,
        )
        node = i * _IB + jax.lax.broadcasted_iota(jnp.int32, (_IB, 1), 0)
        h2 = jnp.where(node < _N, h2, 0.0)
        part = jnp.sum(h2, axis=0, keepdims=True)

        @pl.when(i == 0)
        def _g_init():
            g_acc[...] = jnp.zeros_like(g_acc)

        g_acc[...] += part

        @pl.when(i == ni - 1)
        def _readout():
            g = g_acc[...] / _N
            z = jnp.maximum(
                jax.lax.dot_general(
                    Q_ref[...], g, dimension_numbers=(((1,), (1,)), ((), ())),
                    preferred_element_type=jnp.float32,
                    precision=jax.lax.Precision.HIGHEST,
                ),
                0.0,
            )
            out_ref[...] = jax.lax.dot_general(
                P_ref[...], z, dimension_numbers=(((1,), (0,)), ((), ())),
                preferred_element_type=jnp.float32,
                precision=jax.lax.Precision.HIGHEST,
            )


def kernel(x, adj_mat, U0, b0, U, Q, P):
    h = pl.pallas_call(
        _proj_kernel,
        grid=(5,),
        in_specs=[
            pl.BlockSpec((_N // 5, _D), lambda r: (r, 0)),
            pl.BlockSpec((_D, _D), lambda r: (0, 0)),
            pl.BlockSpec((1, _D), lambda r: (0, 0)),
        ],
        out_specs=pl.BlockSpec((_N // 5, _D), lambda r: (r, 0)),
        out_shape=jax.ShapeDtypeStruct((_N, _D), jnp.float32),
    )(x, U0, b0.reshape(1, _D))

    ni = _NI
    nk = _N // _KB
    out = pl.pallas_call(
        _main_kernel,
        grid=(ni, nk),
        in_specs=[
            pl.BlockSpec((_N, _D), lambda i, k: (0, 0)),
            pl.BlockSpec((_KB, _IB), lambda i, k: (k, i)),
            pl.BlockSpec((_D, _D), lambda i, k: (0, 0)),
            pl.BlockSpec((_D, _D), lambda i, k: (0, 0)),
            pl.BlockSpec((1, _D), lambda i, k: (0, 0)),
        ],
        out_specs=pl.BlockSpec((1, 1), lambda i, k: (0, 0)),
        out_shape=jax.ShapeDtypeStruct((1, 1), jnp.float32),
        scratch_shapes=[
            pltpu.VMEM((_IB, _D), jnp.float32),
            pltpu.VMEM((_IB, 1), jnp.float32),
            pltpu.VMEM((1, _D), jnp.float32),
        ],
    )(h, adj_mat, U, Q, P)
    return out[0, 0]


# speedup vs baseline: 1.4934x; 1.1521x over previous
import jax
import jax.numpy as jnp
from jax.experimental import pallas as pl
from jax.experimental.pallas import tpu as pltpu

_N = 10000
_D = 128
_IB = 2048
_KB = 1000
_NI = -(-_N // _IB)


def _proj_kernel(x_ref, U0_ref, b0_ref, h_ref):
    h_ref[...] = (
        jax.lax.dot_general(
            x_ref[...], U0_ref[...],
            dimension_numbers=(((1,), (0,)), ((), ())),
            preferred_element_type=jnp.float32,
            precision=jax.lax.Precision.HIGHEST,
        )
        + b0_ref[...]
    )


def _main_kernel(h_ref, adj_ref, U_ref, Q_ref, P_ref, out_ref,
                 S_acc, deg_acc, g_acc):
    i = pl.program_id(0)
    k = pl.program_id(1)
    ni = pl.num_programs(0)
    nk = pl.num_programs(1)

    @pl.when(k == 0)
    def _init():
        S_acc[...] = jnp.zeros_like(S_acc)
        deg_acc[...] = jnp.zeros_like(deg_acc)

    a = adj_ref[...].astype(jnp.bfloat16)
    hk = h_ref[pl.ds(k * _KB, _KB), :]
    hk_hi = hk.astype(jnp.bfloat16)
    hk_lo = (hk - hk_hi.astype(jnp.float32)).astype(jnp.bfloat16)
    S_acc[...] += jax.lax.dot_general(
        a, hk_hi, dimension_numbers=(((0,), (0,)), ((), ())),
        preferred_element_type=jnp.float32,
    ) + jax.lax.dot_general(
        a, hk_lo, dimension_numbers=(((0,), (0,)), ((), ())),
        preferred_element_type=jnp.float32,
    )
    deg_acc[...] += jnp.sum(adj_ref[...], axis=0, keepdims=True)

    @pl.when(k == nk - 1)
    def _epilogue():
        S = S_acc[...] / deg_acc[...].T
        h2 = jnp.maximum(
            jax.lax.dot_general(
                S, U_ref[...], dimension_numbers=(((1,), (1,)), ((), ())),
                preferred_element_type=jnp.float32,
                precision=jax.lax.Precision.HIGHEST,
            ),
            0.0,
        )
        node = i * _IB + jax.lax.broadcasted_iota(jnp.int32, (_IB, 1), 0)
        h2 = jnp.where(node < _N, h2, 0.0)
        part = jnp.sum(h2, axis=0, keepdims=True)

        @pl.when(i == 0)
        def _g_init():
            g_acc[...] = jnp.zeros_like(g_acc)

        g_acc[...] += part

        @pl.when(i == ni - 1)
        def _readout():
            g = g_acc[...] / _N
            z = jnp.maximum(
                jax.lax.dot_general(
                    Q_ref[...], g, dimension_numbers=(((1,), (1,)), ((), ())),
                    preferred_element_type=jnp.float32,
                    precision=jax.lax.Precision.HIGHEST,
                ),
                0.0,
            )
            out_ref[...] = jax.lax.dot_general(
                P_ref[...], z, dimension_numbers=(((1,), (0,)), ((), ())),
                preferred_element_type=jnp.float32,
                precision=jax.lax.Precision.HIGHEST,
            )


def kernel(x, adj_mat, U0, b0, U, Q, P):
    h = pl.pallas_call(
        _proj_kernel,
        grid=(5,),
        in_specs=[
            pl.BlockSpec((_N // 5, _D), lambda r: (r, 0)),
            pl.BlockSpec((_D, _D), lambda r: (0, 0)),
            pl.BlockSpec((1, _D), lambda r: (0, 0)),
        ],
        out_specs=pl.BlockSpec((_N // 5, _D), lambda r: (r, 0)),
        out_shape=jax.ShapeDtypeStruct((_N, _D), jnp.float32),
    )(x, U0, b0.reshape(1, _D))

    ni = _NI
    nk = _N // _KB
    out = pl.pallas_call(
        _main_kernel,
        grid=(ni, nk),
        in_specs=[
            pl.BlockSpec((_N, _D), lambda i, k: (0, 0)),
            pl.BlockSpec((_KB, _IB), lambda i, k: (k, i)),
            pl.BlockSpec((_D, _D), lambda i, k: (0, 0)),
            pl.BlockSpec((_D, _D), lambda i, k: (0, 0)),
            pl.BlockSpec((1, _D), lambda i, k: (0, 0)),
        ],
        out_specs=pl.BlockSpec((1, 1), lambda i, k: (0, 0)),
        out_shape=jax.ShapeDtypeStruct((1, 1), jnp.float32),
        scratch_shapes=[
            pltpu.VMEM((_IB, _D), jnp.float32),
            pltpu.VMEM((1, _IB), jnp.float32),
            pltpu.VMEM((1, _D), jnp.float32),
        ],
    )(h, adj_mat, U, Q, P)
    return out[0, 0]


# speedup vs baseline: 1.5753x; 1.0549x over previous
import jax
import jax.numpy as jnp
from jax.experimental import pallas as pl
from jax.experimental.pallas import tpu as pltpu

_N = 10000
_D = 128
_IB = 2048
_KB = 2000
_NI = -(-_N // _IB)


def _proj_kernel(x_ref, U0_ref, b0_ref, h_ref):
    h_ref[...] = (
        jax.lax.dot_general(
            x_ref[...], U0_ref[...],
            dimension_numbers=(((1,), (0,)), ((), ())),
            preferred_element_type=jnp.float32,
            precision=jax.lax.Precision.HIGHEST,
        )
        + b0_ref[...]
    )


def _main_kernel(h_ref, adj_ref, U_ref, Q_ref, P_ref, out_ref,
                 S_acc, deg_acc, g_acc):
    i = pl.program_id(0)
    k = pl.program_id(1)
    ni = pl.num_programs(0)
    nk = pl.num_programs(1)

    @pl.when(k == 0)
    def _init():
        S_acc[...] = jnp.zeros_like(S_acc)
        deg_acc[...] = jnp.zeros_like(deg_acc)

    a = adj_ref[...].astype(jnp.bfloat16)
    hk = h_ref[pl.ds(k * _KB, _KB), :]
    hk_hi = hk.astype(jnp.bfloat16)
    hk_lo = (hk - hk_hi.astype(jnp.float32)).astype(jnp.bfloat16)
    S_acc[...] += jax.lax.dot_general(
        a, hk_hi, dimension_numbers=(((0,), (0,)), ((), ())),
        preferred_element_type=jnp.float32,
    ) + jax.lax.dot_general(
        a, hk_lo, dimension_numbers=(((0,), (0,)), ((), ())),
        preferred_element_type=jnp.float32,
    )
    deg_acc[...] += jnp.sum(adj_ref[...], axis=0, keepdims=True)

    @pl.when(k == nk - 1)
    def _epilogue():
        S = S_acc[...] / deg_acc[...].T
        h2 = jnp.maximum(
            jax.lax.dot_general(
                S, U_ref[...], dimension_numbers=(((1,), (1,)), ((), ())),
                preferred_element_type=jnp.float32,
                precision=jax.lax.Precision.HIGHEST,
            ),
            0.0,
        )
        node = i * _IB + jax.lax.broadcasted_iota(jnp.int32, (_IB, 1), 0)
        h2 = jnp.where(node < _N, h2, 0.0)
        part = jnp.sum(h2, axis=0, keepdims=True)

        @pl.when(i == 0)
        def _g_init():
            g_acc[...] = jnp.zeros_like(g_acc)

        g_acc[...] += part

        @pl.when(i == ni - 1)
        def _readout():
            g = g_acc[...] / _N
            z = jnp.maximum(
                jax.lax.dot_general(
                    Q_ref[...], g, dimension_numbers=(((1,), (1,)), ((), ())),
                    preferred_element_type=jnp.float32,
                    precision=jax.lax.Precision.HIGHEST,
                ),
                0.0,
            )
            out_ref[...] = jax.lax.dot_general(
                P_ref[...], z, dimension_numbers=(((1,), (0,)), ((), ())),
                preferred_element_type=jnp.float32,
                precision=jax.lax.Precision.HIGHEST,
            )


def kernel(x, adj_mat, U0, b0, U, Q, P):
    h = pl.pallas_call(
        _proj_kernel,
        grid=(5,),
        in_specs=[
            pl.BlockSpec((_N // 5, _D), lambda r: (r, 0)),
            pl.BlockSpec((_D, _D), lambda r: (0, 0)),
            pl.BlockSpec((1, _D), lambda r: (0, 0)),
        ],
        out_specs=pl.BlockSpec((_N // 5, _D), lambda r: (r, 0)),
        out_shape=jax.ShapeDtypeStruct((_N, _D), jnp.float32),
    )(x, U0, b0.reshape(1, _D))

    ni = _NI
    nk = _N // _KB
    out = pl.pallas_call(
        _main_kernel,
        grid=(ni, nk),
        in_specs=[
            pl.BlockSpec((_N, _D), lambda i, k: (0, 0)),
            pl.BlockSpec((_KB, _IB), lambda i, k: (k, i)),
            pl.BlockSpec((_D, _D), lambda i, k: (0, 0)),
            pl.BlockSpec((_D, _D), lambda i, k: (0, 0)),
            pl.BlockSpec((1, _D), lambda i, k: (0, 0)),
        ],
        out_specs=pl.BlockSpec((1, 1), lambda i, k: (0, 0)),
        out_shape=jax.ShapeDtypeStruct((1, 1), jnp.float32),
        scratch_shapes=[
            pltpu.VMEM((_IB, _D), jnp.float32),
            pltpu.VMEM((1, _IB), jnp.float32),
            pltpu.VMEM((1, _D), jnp.float32),
        ],
    )(h, adj_mat, U, Q, P)
    return out[0, 0]


# speedup vs baseline: 1.9541x; 1.2404x over previous
import jax
import jax.numpy as jnp
from jax.experimental import pallas as pl
from jax.experimental.pallas import tpu as pltpu

_N = 10000
_D = 128
_IB = 512
_NI = -(-_N // _IB)
_RB = 2048


def _proj_kernel(x_ref, U0_ref, b0_ref, hT_ref):
    h = (
        jax.lax.dot_general(
            x_ref[...], U0_ref[...],
            dimension_numbers=(((1,), (0,)), ((), ())),
            preferred_element_type=jnp.float32,
            precision=jax.lax.Precision.HIGHEST,
        )
        + b0_ref[...]
    )
    h_hi = h.astype(jnp.bfloat16)
    h_lo = (h - h_hi.astype(jnp.float32)).astype(jnp.bfloat16)
    hT_ref[...] = jnp.concatenate([h_hi.T, h_lo.T], axis=0)


def _main_kernel(hT_ref, adj_ref, U_ref, Q_ref, P_ref, out_ref, g_acc):
    i = pl.program_id(0)
    ni = pl.num_programs(0)

    a = adj_ref[...]
    ab = a.astype(jnp.bfloat16)
    S2 = jax.lax.dot_general(
        hT_ref[...], ab, dimension_numbers=(((1,), (0,)), ((), ())),
        preferred_element_type=jnp.float32,
    )
    ST = S2[:_D, :] + S2[_D:, :]
    deg = jnp.sum(a, axis=0, keepdims=True)
    STd = ST / deg
    h2T = jnp.maximum(
        jax.lax.dot_general(
            U_ref[...], STd, dimension_numbers=(((1,), (0,)), ((), ())),
            preferred_element_type=jnp.float32,
            precision=jax.lax.Precision.HIGHEST,
        ),
        0.0,
    )
    node = i * _IB + jax.lax.broadcasted_iota(jnp.int32, (1, _IB), 1)
    h2T = jnp.where(node < _N, h2T, 0.0)
    part = jax.lax.dot_general(
        h2T, jnp.ones((_IB, 1), jnp.float32),
        dimension_numbers=(((1,), (0,)), ((), ())),
        preferred_element_type=jnp.float32,
        precision=jax.lax.Precision.HIGHEST,
    )

    @pl.when(i == 0)
    def _g_init():
        g_acc[...] = jnp.zeros_like(g_acc)

    g_acc[...] += part

    @pl.when(i == ni - 1)
    def _readout():
        g = g_acc[...] / _N
        z = jnp.maximum(
            jax.lax.dot_general(
                Q_ref[...], g, dimension_numbers=(((1,), (0,)), ((), ())),
                preferred_element_type=jnp.float32,
                precision=jax.lax.Precision.HIGHEST,
            ),
            0.0,
        )
        out_ref[...] = jax.lax.dot_general(
            P_ref[...], z, dimension_numbers=(((1,), (0,)), ((), ())),
            preferred_element_type=jnp.float32,
            precision=jax.lax.Precision.HIGHEST,
        )


def kernel(x, adj_mat, U0, b0, U, Q, P):
    hT = pl.pallas_call(
        _proj_kernel,
        grid=(-(-_N // _RB),),
        in_specs=[
            pl.BlockSpec((_RB, _D), lambda r: (r, 0)),
            pl.BlockSpec((_D, _D), lambda r: (0, 0)),
            pl.BlockSpec((1, _D), lambda r: (0, 0)),
        ],
        out_specs=pl.BlockSpec((2 * _D, _RB), lambda r: (0, r)),
        out_shape=jax.ShapeDtypeStruct((2 * _D, _N), jnp.bfloat16),
    )(x, U0, b0.reshape(1, _D))

    out = pl.pallas_call(
        _main_kernel,
        grid=(_NI,),
        in_specs=[
            pl.BlockSpec((2 * _D, _N), lambda i: (0, 0)),
            pl.BlockSpec((_N, _IB), lambda i: (0, i)),
            pl.BlockSpec((_D, _D), lambda i: (0, 0)),
            pl.BlockSpec((_D, _D), lambda i: (0, 0)),
            pl.BlockSpec((1, _D), lambda i: (0, 0)),
        ],
        out_specs=pl.BlockSpec((1, 1), lambda i: (0, 0)),
        out_shape=jax.ShapeDtypeStruct((1, 1), jnp.float32),
        scratch_shapes=[
            pltpu.VMEM((_D, 1), jnp.float32),
        ],
    )(hT, adj_mat, U, Q, P)
    return out[0, 0]
